# baseline (speedup 1.0000x reference)
import numpy as np

import jax
import jax.numpy as jnp
from jax.experimental import pallas as pl
from jax.experimental.pallas import tpu as pltpu
from jax.sharding import Mesh, PartitionSpec as P

LLM_D = 2048
VIT_D = 1024
POOL = 64
TOPK = 8
ALPHA = 16
IN_F = 2048
OUT_F = 2048
TOK = 8192

TILE = 512
SCALING = ALPHA / TOPK
K_RATIO = VIT_D / LLM_D


def _route_fold_kernel(ql_ref, qv_ref, kl_ref, kv_ref, a_ref, b_pool_ref,
                       w_ref, weff_ref):
    hi = jax.lax.Precision.HIGHEST
    s_llm = jax.lax.dot_general(ql_ref[...], kl_ref[...],
                                (((1,), (1,)), ((), ())), precision=hi)
    s_vit = jax.lax.dot_general(qv_ref[...], kv_ref[...],
                                (((1,), (1,)), ((), ())), precision=hi)
    s_row = s_llm + K_RATIO * s_vit
    s_col = jnp.reshape(s_row, (POOL, 1))
    j_idx = jax.lax.broadcasted_iota(jnp.int32, (POOL, POOL), 1)
    k_idx = jax.lax.broadcasted_iota(jnp.int32, (POOL, POOL), 0)
    beats = (s_row > s_col) | ((s_row == s_col) & (j_idx < k_idx))
    rank = jnp.sum(beats.astype(jnp.int32), axis=1, keepdims=True)
    mask = (rank < TOPK).astype(jnp.float32)
    a_m = a_ref[...] * (jnp.reshape(mask, (1, POOL)) * SCALING)
    delta = jax.lax.dot_general(b_pool_ref[...], a_m,
                                (((0,), (1,)), ((), ())))
    weff_ref[...] = w_ref[...] + delta


def _main_kernel(x_ref, weff_ref, bias_ref, o_ref):
    out = jax.lax.dot_general(x_ref[...], weff_ref[...],
                              (((1,), (1,)), ((), ())))
    o_ref[...] = out + bias_ref[...]


def _shard_impl(x, ql, qv, kl, kv, a_pool, b_pool, w, bias):
    tok = x.shape[0]
    w_eff = pl.pallas_call(
        _route_fold_kernel,
        out_shape=jax.ShapeDtypeStruct((OUT_F, IN_F), jnp.float32),
    )(ql, qv, kl, kv, a_pool, b_pool, w)

    full = lambda shape: pl.BlockSpec(shape, lambda i: (0, 0))
    return pl.pallas_call(
        _main_kernel,
        grid=(tok // TILE,),
        in_specs=[
            pl.BlockSpec((TILE, IN_F), lambda i: (i, 0)),
            full((OUT_F, IN_F)),
            full((1, OUT_F)),
        ],
        out_specs=pl.BlockSpec((TILE, OUT_F), lambda i: (i, 0)),
        out_shape=jax.ShapeDtypeStruct((tok, OUT_F), jnp.float32),
        compiler_params=pltpu.CompilerParams(
            dimension_semantics=("arbitrary",),
        ),
    )(x, w_eff, bias)


@jax.jit
def kernel(x, llm_query, vit_query, static_keys_llm, static_keys_vit,
           A_pool, B_pool, W, b):
    ql = jnp.reshape(llm_query, (1, LLM_D))
    qv = jnp.reshape(vit_query, (1, VIT_D))
    bias = jnp.reshape(b, (1, OUT_F))

    devs = jax.devices()
    n_shards = 1
    for c in (8, 4, 2):
        if len(devs) >= c and (TOK // TILE) % c == 0:
            n_shards = c
            break

    if n_shards == 1:
        return _shard_impl(x, ql, qv, static_keys_llm, static_keys_vit,
                           A_pool, B_pool, W, bias)

    mesh = Mesh(np.array(devs[:n_shards]), ("d",))
    rep = P(None, None)
    fn = jax.shard_map(
        _shard_impl,
        mesh=mesh,
        in_specs=(P("d", None), rep, rep, rep, rep, rep, rep, rep, rep),
        out_specs=P("d", None),
        check_vma=False,
    )
    return fn(x, ql, qv, static_keys_llm, static_keys_vit,
              A_pool, B_pool, W, bias)


# speedup vs baseline: 5.9656x; 5.9656x over previous
import jax
import jax.numpy as jnp
from jax.experimental import pallas as pl
from jax.experimental.pallas import tpu as pltpu

LLM_D = 2048
VIT_D = 1024
POOL = 64
TOPK = 8
ALPHA = 16
IN_F = 2048
OUT_F = 2048
TOK = 8192

TILE = 1024
SCALING = ALPHA / TOPK
K_RATIO = VIT_D / LLM_D


def _route_fold_kernel(ql_ref, qv_ref, kl_ref, kv_ref, a_ref, b_pool_ref,
                       w_ref, weff_ref):
    hi = jax.lax.Precision.HIGHEST
    s_llm = jax.lax.dot_general(ql_ref[...], kl_ref[...],
                                (((1,), (1,)), ((), ())), precision=hi)
    s_vit = jax.lax.dot_general(qv_ref[...], kv_ref[...],
                                (((1,), (1,)), ((), ())), precision=hi)
    s_row = s_llm + K_RATIO * s_vit
    s_col = jnp.reshape(s_row, (POOL, 1))
    j_idx = jax.lax.broadcasted_iota(jnp.int32, (POOL, POOL), 1)
    k_idx = jax.lax.broadcasted_iota(jnp.int32, (POOL, POOL), 0)
    beats = (s_row > s_col) | ((s_row == s_col) & (j_idx < k_idx))
    rank = jnp.sum(beats.astype(jnp.int32), axis=1, keepdims=True)
    mask = (rank < TOPK).astype(jnp.float32)
    a_m = a_ref[...] * (jnp.reshape(mask, (1, POOL)) * SCALING)
    delta = jax.lax.dot_general(b_pool_ref[...], a_m,
                                (((0,), (1,)), ((), ())))
    weff_ref[...] = (w_ref[...] + delta).astype(jnp.bfloat16)


def _main_kernel(x_ref, weff_ref, bias_ref, o_ref):
    out = jax.lax.dot_general(x_ref[...], weff_ref[...],
                              (((1,), (1,)), ((), ())),
                              preferred_element_type=jnp.float32)
    o_ref[...] = out + bias_ref[...]


@jax.jit
def kernel(x, llm_query, vit_query, static_keys_llm, static_keys_vit,
           A_pool, B_pool, W, b):
    ql = jnp.reshape(llm_query, (1, LLM_D))
    qv = jnp.reshape(vit_query, (1, VIT_D))
    bias = jnp.reshape(b, (1, OUT_F))

    w_eff = pl.pallas_call(
        _route_fold_kernel,
        out_shape=jax.ShapeDtypeStruct((OUT_F, IN_F), jnp.bfloat16),
    )(ql, qv, static_keys_llm, static_keys_vit, A_pool, B_pool, W)

    full = lambda shape: pl.BlockSpec(shape, lambda i: (0, 0))
    return pl.pallas_call(
        _main_kernel,
        grid=(TOK // TILE,),
        in_specs=[
            pl.BlockSpec((TILE, IN_F), lambda i: (i, 0)),
            full((OUT_F, IN_F)),
            full((1, OUT_F)),
        ],
        out_specs=pl.BlockSpec((TILE, OUT_F), lambda i: (i, 0)),
        out_shape=jax.ShapeDtypeStruct((TOK, OUT_F), jnp.float32),
        compiler_params=pltpu.CompilerParams(
            dimension_semantics=("arbitrary",),
        ),
    )(x, w_eff, bias)


# speedup vs baseline: 6.3621x; 1.0665x over previous
import jax
import jax.numpy as jnp
from jax.experimental import pallas as pl
from jax.experimental.pallas import tpu as pltpu

LLM_D = 2048
VIT_D = 1024
POOL = 64
TOPK = 8
ALPHA = 16
IN_F = 2048
OUT_F = 2048
TOK = 8192

TILE = 512
SCALING = ALPHA / TOPK
K_RATIO = VIT_D / LLM_D


def _fused_kernel(x_ref, ql_ref, qv_ref, kl_ref, kv_ref, a_ref, b_pool_ref,
                  w_ref, bias_ref, o_ref, weff_ref):
    @pl.when(pl.program_id(0) == 0)
    def _fold():
        hi = jax.lax.Precision.HIGHEST
        s_llm = jax.lax.dot_general(ql_ref[...], kl_ref[...],
                                    (((1,), (1,)), ((), ())), precision=hi)
        s_vit = jax.lax.dot_general(qv_ref[...], kv_ref[...],
                                    (((1,), (1,)), ((), ())), precision=hi)
        s_row = s_llm + K_RATIO * s_vit
        s_col = jnp.reshape(s_row, (POOL, 1))
        j_idx = jax.lax.broadcasted_iota(jnp.int32, (POOL, POOL), 1)
        k_idx = jax.lax.broadcasted_iota(jnp.int32, (POOL, POOL), 0)
        beats = (s_row > s_col) | ((s_row == s_col) & (j_idx < k_idx))
        rank = jnp.sum(beats.astype(jnp.int32), axis=1, keepdims=True)
        mask = (rank < TOPK).astype(jnp.float32)
        a_m = a_ref[...] * (jnp.reshape(mask, (1, POOL)) * SCALING)
        delta = jax.lax.dot_general(b_pool_ref[...], a_m,
                                    (((0,), (1,)), ((), ())))
        weff_ref[...] = (w_ref[...] + delta).astype(jnp.bfloat16)

    out = jax.lax.dot_general(x_ref[...], weff_ref[...],
                              (((1,), (1,)), ((), ())),
                              preferred_element_type=jnp.float32)
    o_ref[...] = out + bias_ref[...]


@jax.jit
def kernel(x, llm_query, vit_query, static_keys_llm, static_keys_vit,
           A_pool, B_pool, W, b):
    ql = jnp.reshape(llm_query, (1, LLM_D))
    qv = jnp.reshape(vit_query, (1, VIT_D))
    bias = jnp.reshape(b, (1, OUT_F))

    full = lambda shape: pl.BlockSpec(shape, lambda i: (0, 0))
    return pl.pallas_call(
        _fused_kernel,
        grid=(TOK // TILE,),
        in_specs=[
            pl.BlockSpec((TILE, IN_F), lambda i: (i, 0)),
            full((1, LLM_D)),
            full((1, VIT_D)),
            full((POOL, LLM_D)),
            full((POOL, VIT_D)),
            full((IN_F, POOL)),
            full((POOL, OUT_F)),
            full((OUT_F, IN_F)),
            full((1, OUT_F)),
        ],
        out_specs=pl.BlockSpec((TILE, OUT_F), lambda i: (i, 0)),
        out_shape=jax.ShapeDtypeStruct((TOK, OUT_F), jnp.float32),
        scratch_shapes=[pltpu.VMEM((OUT_F, IN_F), jnp.bfloat16)],
        compiler_params=pltpu.CompilerParams(
            dimension_semantics=("arbitrary",),
        ),
    )(x, ql, qv, static_keys_llm, static_keys_vit, A_pool, B_pool, W, bias)
